# baseline (speedup 1.0000x reference)
import functools

import jax
import jax.numpy as jnp
from jax import lax
from jax.experimental import pallas as pl
from jax.experimental.pallas import tpu as pltpu
from jax.experimental.pallas import tpu_sc as plsc

N = 10000
E = 320000
FDIM = 128
K = 64
SR_CUT = 10.0
NB = 3
NRI = 2
NRA = 2
NRO = 1
LN2 = 0.6931471805599453

NC, NS, L = 2, 16, 16
NW = NC * NS
CE = 128
NCH = E // CE
ROWS_Q, ROWS_R = divmod(NCH, NW)
EPW_MAX = (ROWS_Q + 1) * CE
CE2 = 64
NCH2 = E // CE2
PQ, PR = divmod(NCH2 // 2, NW)
NP = 10112
NPS = NP // NS
BN = 2000
GE = 2000


def _ssp(v):
    return jnp.maximum(v, 0.0) + jnp.log1p(jnp.exp(-jnp.abs(v))) - LN2


def _softplus(v):
    return jnp.maximum(v, 0.0) + jnp.log1p(jnp.exp(-jnp.abs(v)))


def _d2_body(rx_hbm, ry_hbm, rz_hbm, ii_hbm, ij_hbm, d2_hbm,
             rx, ry, rz, ii, ij, d2):
    wid = lax.axis_index("s") * NC + lax.axis_index("c")
    base = (wid * ROWS_Q + jnp.minimum(wid, ROWS_R)) * CE
    extra = wid < ROWS_R
    nmain = ROWS_Q * CE
    pltpu.sync_copy(rx_hbm, rx)
    pltpu.sync_copy(ry_hbm, ry)
    pltpu.sync_copy(rz_hbm, rz)
    pltpu.sync_copy(ii_hbm.at[pl.ds(base, nmain)], ii.at[pl.ds(0, nmain)])
    pltpu.sync_copy(ij_hbm.at[pl.ds(base, nmain)], ij.at[pl.ds(0, nmain)])

    @pl.when(extra)
    def _():
        pltpu.sync_copy(ii_hbm.at[pl.ds(base + nmain, CE)],
                        ii.at[pl.ds(nmain, CE)])
        pltpu.sync_copy(ij_hbm.at[pl.ds(base + nmain, CE)],
                        ij.at[pl.ds(nmain, CE)])

    def body(k, _):
        sl = pl.ds(k * L, L)
        a = ii[sl]
        b = ij[sl]
        dx = plsc.load_gather(rx, [a]) - plsc.load_gather(rx, [b])
        dy = plsc.load_gather(ry, [a]) - plsc.load_gather(ry, [b])
        dz = plsc.load_gather(rz, [a]) - plsc.load_gather(rz, [b])
        d2[sl] = dx * dx + dy * dy + dz * dz
        return 0

    nedge = nmain + jnp.where(extra, CE, 0)
    lax.fori_loop(0, nedge // L, body, 0)
    pltpu.sync_copy(d2.at[pl.ds(0, nmain)], d2_hbm.at[pl.ds(base, nmain)])

    @pl.when(extra)
    def _():
        pltpu.sync_copy(d2.at[pl.ds(nmain, CE)],
                        d2_hbm.at[pl.ds(base + nmain, CE)])


def _make_d2():
    mesh = plsc.VectorSubcoreMesh(core_axis_name="c", subcore_axis_name="s",
                                  num_cores=NC, num_subcores=NS)
    return pl.kernel(
        _d2_body,
        out_type=jax.ShapeDtypeStruct((E,), jnp.float32),
        mesh=mesh,
        scratch_types=[
            pltpu.VMEM((N,), jnp.float32),
            pltpu.VMEM((N,), jnp.float32),
            pltpu.VMEM((N,), jnp.float32),
            pltpu.VMEM((EPW_MAX,), jnp.int32),
            pltpu.VMEM((EPW_MAX,), jnp.int32),
            pltpu.VMEM((EPW_MAX,), jnp.float32),
        ],
        compiler_params=pltpu.CompilerParams(needs_layout_passes=False),
    )


def _seg_body(g3, hj, ii_hbm, ij_hbm, macc_hbm,
              idxi0, idxi1, idxj0, idxj1, hjg0, hjg1, gb0, gb1, xj0, xj1,
              acc, gsem0, gsem1, csem0, csem1, ssem0, ssem1,
              isem0, isem1, ksem0, ksem1):
    cid = lax.axis_index("c")
    sid = lax.axis_index("s")
    wid = sid * NC + cid
    idxis, idxjs = (idxi0, idxi1), (idxj0, idxj1)
    hjgs, gbs, xjs = (hjg0, hjg1), (gb0, gb1), (xj0, xj1)
    gsems, csems, ssems = (gsem0, gsem1), (csem0, csem1), (ssem0, ssem1)
    isems, ksems = (isem0, isem1), (ksem0, ksem1)

    base = (wid * PQ + jnp.minimum(wid, PR)) * 2
    npairs = PQ + jnp.where(wid < PR, 1, 0)

    def zrow(r, _):
        for c in range(FDIM // L):
            xj0[r, pl.ds(c * L, L)] = jnp.zeros((L,), jnp.float32)
        return 0

    lax.fori_loop(0, CE2, zrow, 0)
    r0 = sid * NPS
    for j0 in range(0, NPS - CE2 + 1, CE2):
        pltpu.sync_copy(xj0, acc.at[pl.ds(r0 + j0, CE2)])
    rem = NPS % CE2
    if rem:
        pltpu.sync_copy(xj0.at[pl.ds(0, rem)],
                        acc.at[pl.ds(r0 + NPS - rem, rem)])
    plsc.subcore_barrier()

    for b in range(2):
        pltpu.sync_copy(ij_hbm.at[pl.ds((base + b) * CE2, CE2)], idxjs[b])
        pltpu.async_copy(hj.at[idxjs[b]], hjgs[b], gsems[b])
        pltpu.async_copy(g3.at[base + b], gbs[b], csems[b])
        pltpu.async_copy(ii_hbm.at[pl.ds((base + b) * CE2, CE2)],
                         idxis[b], ksems[b])

    def pair(m, _):
        for b in range(2):
            k = base + 2 * m + b
            pltpu.make_async_copy(hj.at[idxjs[b]], hjgs[b], gsems[b]).wait()

            @pl.when(m + 1 < npairs)
            def _():
                pltpu.async_copy(ij_hbm.at[pl.ds((k + 2) * CE2, CE2)],
                                 idxjs[b], isems[b])

            pltpu.make_async_copy(g3.at[k], gbs[b], csems[b]).wait()

            @pl.when(m > 0)
            def _():
                pltpu.make_async_copy(xjs[b], acc.at[idxis[b]],
                                      ssems[b]).wait()
                pltpu.async_copy(ii_hbm.at[pl.ds(k * CE2, CE2)],
                                 idxis[b], ksems[b])

            def mrow(rr, _):
                for c2 in range(FDIM // 32):
                    gp = gbs[b][rr, pl.ds(c2 * 32, 32)]
                    ga, gb_ = plsc.unpack(
                        gp, format=plsc.PackFormat.INTERLEAVED)
                    s0 = pl.ds(c2 * 32, L)
                    s1 = pl.ds(c2 * 32 + L, L)
                    xjs[b][rr, s0] = hjgs[b][rr, s0] * ga
                    xjs[b][rr, s1] = hjgs[b][rr, s1] * gb_
                return 0

            lax.fori_loop(0, CE2, mrow, 0)
            pltpu.make_async_copy(ii_hbm.at[pl.ds(k * CE2, CE2)],
                                  idxis[b], ksems[b]).wait()
            pltpu.async_copy(xjs[b], acc.at[idxis[b]], ssems[b], add=True)

            @pl.when(m + 1 < npairs)
            def _():
                pltpu.make_async_copy(ij_hbm.at[pl.ds((k + 2) * CE2, CE2)],
                                      idxjs[b], isems[b]).wait()
                pltpu.async_copy(hj.at[idxjs[b]], hjgs[b], gsems[b])
                pltpu.async_copy(g3.at[k + 2], gbs[b], csems[b])
        return 0

    lax.fori_loop(0, npairs, pair, 0)
    for b in range(2):
        pltpu.make_async_copy(xjs[b], acc.at[idxis[b]], ssems[b]).wait()
    plsc.subcore_barrier()
    pltpu.sync_copy(acc.at[pl.ds(r0, NPS)],
                    macc_hbm.at[cid, pl.ds(r0, NPS)])


def _make_seg():
    mesh = plsc.VectorSubcoreMesh(core_axis_name="c", subcore_axis_name="s",
                                  num_cores=NC, num_subcores=NS)
    return pl.kernel(
        _seg_body,
        out_type=jax.ShapeDtypeStruct((NC, NP, FDIM), jnp.float32),
        mesh=mesh,
        scratch_types=[
            pltpu.VMEM((CE2,), jnp.int32),
            pltpu.VMEM((CE2,), jnp.int32),
            pltpu.VMEM((CE2,), jnp.int32),
            pltpu.VMEM((CE2,), jnp.int32),
            pltpu.VMEM((CE2, FDIM), jnp.float32),
            pltpu.VMEM((CE2, FDIM), jnp.float32),
            pltpu.VMEM((CE2, FDIM), jnp.bfloat16),
            pltpu.VMEM((CE2, FDIM), jnp.bfloat16),
            pltpu.VMEM((CE2, FDIM), jnp.float32),
            pltpu.VMEM((CE2, FDIM), jnp.float32),
            pltpu.VMEM_SHARED((NP, FDIM), jnp.float32),
        ] + [pltpu.SemaphoreType.DMA] * 10,
        compiler_params=pltpu.CompilerParams(needs_layout_passes=False,
                                             use_tc_tiling_on_sc=False),
    )


def _g_body(d2_ref, cen_ref, wid_ref, k2f_ref, g0_ref, g1_ref, g2_ref,
            dij_ref):
    d2 = d2_ref[...]
    dij = jnp.sqrt(jnp.maximum(d2, 0.0))
    dij_ref[...] = dij
    xr = dij * (1.0 / SR_CUT)
    xr2 = xr * xr
    xr3 = xr2 * xr
    cut = 1.0 + ((15.0 - 6.0 * xr) * xr - 10.0) * xr3
    cut = jnp.where(dij < SR_CUT, cut, 0.0)
    mu = _softplus(cen_ref[...])
    beta = _softplus(wid_ref[...])
    diff = jnp.exp(-dij) - mu
    rbf = cut * jnp.exp(-beta * diff * diff)
    for b, ref in enumerate((g0_ref, g1_ref, g2_ref)):
        ref[...] = jnp.dot(rbf, k2f_ref[b],
                           preferred_element_type=jnp.float32
                           ).astype(jnp.bfloat16)


def _run_g(d2c, centers_r, widths_r, k2fP):
    return pl.pallas_call(
        _g_body,
        grid=(E // GE,),
        in_specs=[
            pl.BlockSpec((GE, 1), lambda i: (i, 0)),
            pl.BlockSpec((1, K), lambda i: (0, 0)),
            pl.BlockSpec((1, K), lambda i: (0, 0)),
            pl.BlockSpec((NB, K, FDIM), lambda i: (0, 0, 0)),
        ],
        out_specs=[pl.BlockSpec((GE, FDIM), lambda i: (i, 0))] * 3 +
        [pl.BlockSpec((GE, 1), lambda i: (i, 0))],
        out_shape=[jax.ShapeDtypeStruct((E, FDIM), jnp.bfloat16)] * 3 +
        [jax.ShapeDtypeStruct((E, 1), jnp.float32)],
    )(d2c, centers_r, widths_r, k2fP)


def _c0_body(z_ref, emb_ref, wi_ref, bi_ref, wj_ref, bj_ref,
             x_ref, xi_ref, hj_ref):
    zb = z_ref[...]
    iota = lax.broadcasted_iota(jnp.int32, (BN, FDIM), 1)
    oh = (iota == zb).astype(jnp.float32)
    x = jnp.dot(oh, emb_ref[...], preferred_element_type=jnp.float32)
    x_ref[...] = x
    xa = _ssp(x)
    xi_ref[...] = jnp.dot(xa, wi_ref[...],
                          preferred_element_type=jnp.float32) + bi_ref[...]
    hj_ref[...] = jnp.dot(xa, wj_ref[...],
                          preferred_element_type=jnp.float32) + bj_ref[...]


def _run_c0(z2, emb_pad, wi, bi, wj, bj):
    full = lambda shape: pl.BlockSpec(shape, lambda i: (0,) * len(shape))
    return pl.pallas_call(
        _c0_body,
        grid=(N // BN,),
        in_specs=[
            pl.BlockSpec((BN, 1), lambda i: (i, 0)),
            full((FDIM, FDIM)),
            full((FDIM, FDIM)),
            full((1, FDIM)),
            full((FDIM, FDIM)),
            full((1, FDIM)),
        ],
        out_specs=[pl.BlockSpec((BN, FDIM), lambda i: (i, 0))] * 3,
        out_shape=[jax.ShapeDtypeStruct((N, FDIM), jnp.float32)] * 3,
    )(z2, emb_pad, wi, bi, wj, bj)


def _node_body(has_next, xi_ref, macc_ref, x_ref,
               riW1_ref, rib1_ref, riW2_ref, rib2_ref,
               projW_ref, projb_ref, u_ref,
               raW1_ref, rab1_ref, raW2_ref, rab2_ref,
               roW1_ref, rob1_ref, roW2_ref, rob2_ref,
               outW_ref, outb_ref, wiN_ref, biN_ref, wjN_ref, bjN_ref,
               *out_refs):
    dot = functools.partial(jnp.dot, preferred_element_type=jnp.float32)
    m = xi_ref[...] + macc_ref[0] + macc_ref[1]
    for r in range(NRI):
        y = _ssp(dot(_ssp(m), riW1_ref[r]) + rib1_ref[r])
        m = m + dot(y, riW2_ref[r]) + rib2_ref[r]
    m = _ssp(m)
    x = u_ref[...] * x_ref[...] + dot(m, projW_ref[...]) + projb_ref[...]
    for r in range(NRA):
        y = _ssp(dot(_ssp(x), raW1_ref[r]) + rab1_ref[r])
        x = x + dot(y, raW2_ref[r]) + rab2_ref[r]
    o = x
    for r in range(NRO):
        y = _ssp(dot(_ssp(o), roW1_ref[r]) + rob1_ref[r])
        o = o + dot(y, roW2_ref[r]) + rob2_ref[r]
    out_refs[0][...] = dot(_ssp(o), outW_ref[...]) + outb_ref[...]
    if has_next:
        out_refs[1][...] = x
        xa = _ssp(x)
        out_refs[2][...] = dot(xa, wiN_ref[...]) + biN_ref[...]
        out_refs[3][...] = dot(xa, wjN_ref[...]) + bjN_ref[...]


def _run_node(has_next, xi, macc, x, wts):
    full = lambda shape: pl.BlockSpec(shape, lambda i: (0,) * len(shape))
    wspecs = [
        full((NRI, FDIM, FDIM)), full((NRI, 1, FDIM)),
        full((NRI, FDIM, FDIM)), full((NRI, 1, FDIM)),
        full((FDIM, FDIM)), full((1, FDIM)), full((1, FDIM)),
        full((NRA, FDIM, FDIM)), full((NRA, 1, FDIM)),
        full((NRA, FDIM, FDIM)), full((NRA, 1, FDIM)),
        full((NRO, FDIM, FDIM)), full((NRO, 1, FDIM)),
        full((NRO, FDIM, FDIM)), full((NRO, 1, FDIM)),
        full((FDIM, 2)), full((1, 2)),
        full((FDIM, FDIM)), full((1, FDIM)),
        full((FDIM, FDIM)), full((1, FDIM)),
    ]
    out_specs = [pl.BlockSpec((BN, 2), lambda i: (i, 0))]
    out_shape = [jax.ShapeDtypeStruct((N, 2), jnp.float32)]
    if has_next:
        out_specs += [pl.BlockSpec((BN, FDIM), lambda i: (i, 0))] * 3
        out_shape += [jax.ShapeDtypeStruct((N, FDIM), jnp.float32)] * 3
    return pl.pallas_call(
        functools.partial(_node_body, has_next),
        grid=(N // BN,),
        in_specs=[
            pl.BlockSpec((BN, FDIM), lambda i: (i, 0)),
            pl.BlockSpec((NC, BN, FDIM), lambda i: (0, i, 0)),
            pl.BlockSpec((BN, FDIM), lambda i: (i, 0)),
        ] + wspecs,
        out_specs=out_specs,
        out_shape=out_shape,
    )(xi, macc, x, *wts)


def _fin_body(z_ref, o0_ref, o1_ref, o2_ref,
              esc_ref, esh_ref, qsc_ref, qsh_ref,
              ea_ref, qa_ref, nh_ref):
    i = pl.program_id(0)
    zb = z_ref[...]
    iota = lax.broadcasted_iota(jnp.int32, (BN, FDIM), 1)
    oh = (iota == zb).astype(jnp.float32)
    dot = functools.partial(jnp.dot, preferred_element_type=jnp.float32)
    o0 = o0_ref[...]
    o1 = o1_ref[...]
    o2 = o2_ref[...]
    s = o0 + o1 + o2
    ea_ref[...] = dot(oh, esc_ref[...]) * s[:, 0:1] + dot(oh, esh_ref[...])
    qa_ref[...] = dot(oh, qsc_ref[...]) * s[:, 1:2] + dot(oh, qsh_ref[...])
    p0 = o0 * o0
    p1 = o1 * o1
    p2 = o2 * o2
    part = jnp.sum(p1 / (p1 + p0 + 1e-07)) + jnp.sum(p2 / (p2 + p1 + 1e-07))

    @pl.when(i == 0)
    def _():
        nh_ref[...] = jnp.zeros((1, 1), jnp.float32)

    nh_ref[...] += part * (1.0 / (2.0 * N))


def _run_fin(z2, o0, o1, o2, esc, esh, qsc, qsh):
    full = lambda shape: pl.BlockSpec(shape, lambda i: (0,) * len(shape))
    return pl.pallas_call(
        _fin_body,
        grid=(N // BN,),
        in_specs=[
            pl.BlockSpec((BN, 1), lambda i: (i, 0)),
            pl.BlockSpec((BN, 2), lambda i: (i, 0)),
            pl.BlockSpec((BN, 2), lambda i: (i, 0)),
            pl.BlockSpec((BN, 2), lambda i: (i, 0)),
            full((FDIM, 1)), full((FDIM, 1)), full((FDIM, 1)), full((FDIM, 1)),
        ],
        out_specs=[
            pl.BlockSpec((BN, 1), lambda i: (i, 0)),
            pl.BlockSpec((BN, 1), lambda i: (i, 0)),
            pl.BlockSpec((1, 1), lambda i: (0, 0)),
        ],
        out_shape=[
            jax.ShapeDtypeStruct((N, 1), jnp.float32),
            jax.ShapeDtypeStruct((N, 1), jnp.float32),
            jax.ShapeDtypeStruct((1, 1), jnp.float32),
        ],
    )(z2, o0, o1, o2, esc, esh, qsc, qsh)


def kernel(Z, R, idx_i, idx_j, params):
    p = params
    idx_i = idx_i.astype(jnp.int32)
    idx_j = idx_j.astype(jnp.int32)
    z2 = Z.astype(jnp.int32).reshape(N, 1)
    rx, ry, rz = R[:, 0], R[:, 1], R[:, 2]

    d2 = _make_d2()(rx, ry, rz, idx_i, idx_j)
    d2c = d2.reshape(E, 1)

    centers_r = p['centers'].reshape(1, K)
    widths_r = p['widths'].reshape(1, K)
    emb_pad = jnp.zeros((FDIM, FDIM), jnp.float32).at[:95].set(p['emb'])
    pad1 = lambda t: jnp.zeros((FDIM, 1), jnp.float32).at[:95, 0].set(t)

    seg = _make_seg()

    src = []
    for c2 in range(FDIM // 32):
        for i in range(16):
            src += [c2 * 32 + i, c2 * 32 + 16 + i]
    k2fP = p['k2f'][:, :, jnp.array(src, jnp.int32)]

    x, xi, hj = _run_c0(
        z2, emb_pad, p['Wi'][0], p['bi'][0].reshape(1, FDIM),
        p['Wj'][0], p['bj'][0].reshape(1, FDIM))

    gs0, gs1, gs2, dij_c = _run_g(d2c, centers_r, widths_r, k2fP)
    gs = (gs0, gs1, gs2)

    outs = []
    for b in range(NB):
        g3 = gs[b].reshape(NCH2, CE2, FDIM)
        macc = seg(g3, hj, idx_i, idx_j)
        has_next = b < NB - 1
        nb = b + 1 if has_next else 0
        wts = [
            p['riW1'][b], p['rib1'][b].reshape(NRI, 1, FDIM),
            p['riW2'][b], p['rib2'][b].reshape(NRI, 1, FDIM),
            p['projW'][b], p['projb'][b].reshape(1, FDIM),
            p['u'][b].reshape(1, FDIM),
            p['raW1'][b], p['rab1'][b].reshape(NRA, 1, FDIM),
            p['raW2'][b], p['rab2'][b].reshape(NRA, 1, FDIM),
            p['roW1'][b], p['rob1'][b].reshape(NRO, 1, FDIM),
            p['roW2'][b], p['rob2'][b].reshape(NRO, 1, FDIM),
            p['outW'][b], p['outb'][b].reshape(1, 2),
            p['Wi'][nb], p['bi'][nb].reshape(1, FDIM),
            p['Wj'][nb], p['bj'][nb].reshape(1, FDIM),
        ]
        res = _run_node(has_next, xi, macc, x, wts)
        if has_next:
            out_b, x, xi, hj = res
        else:
            (out_b,) = res
        outs.append(out_b)

    ea, qa, nh = _run_fin(z2, outs[0], outs[1], outs[2],
                          pad1(p['Escale']), pad1(p['Eshift']),
                          pad1(p['Qscale']), pad1(p['Qshift']))
    return (ea.reshape(N), qa.reshape(N), dij_c.reshape(E), nh.reshape(()))


# speedup vs baseline: 1.4319x; 1.4319x over previous
import functools

import jax
import jax.numpy as jnp
from jax import lax
from jax.experimental import pallas as pl
from jax.experimental.pallas import tpu as pltpu
from jax.experimental.pallas import tpu_sc as plsc

N = 10000
E = 320000
FDIM = 128
K = 64
SR_CUT = 10.0
NB = 3
NRI = 2
NRA = 2
NRO = 1
LN2 = 0.6931471805599453

NC, NS, L = 2, 16, 16
NW = NC * NS
CE = 128
NCH = E // CE
ROWS_Q, ROWS_R = divmod(NCH, NW)
EPW_MAX = (ROWS_Q + 1) * CE
CE2 = 64
NCH2 = E // CE2
PQ, PR = divmod(NCH2 // 2, NW)
NP = 10112
NPS = NP // NS
BN = 2000
GE = 2000


def _ssp(v):
    return jnp.maximum(v, 0.0) + jnp.log1p(jnp.exp(-jnp.abs(v))) - LN2


def _softplus(v):
    return jnp.maximum(v, 0.0) + jnp.log1p(jnp.exp(-jnp.abs(v)))


def _d2_body(rx_hbm, ry_hbm, rz_hbm, ii_hbm, ij_hbm, d2_hbm,
             rx, ry, rz, ii, ij, d2):
    wid = lax.axis_index("s") * NC + lax.axis_index("c")
    base = (wid * ROWS_Q + jnp.minimum(wid, ROWS_R)) * CE
    extra = wid < ROWS_R
    nmain = ROWS_Q * CE
    pltpu.sync_copy(rx_hbm, rx)
    pltpu.sync_copy(ry_hbm, ry)
    pltpu.sync_copy(rz_hbm, rz)
    pltpu.sync_copy(ii_hbm.at[pl.ds(base, nmain)], ii.at[pl.ds(0, nmain)])
    pltpu.sync_copy(ij_hbm.at[pl.ds(base, nmain)], ij.at[pl.ds(0, nmain)])

    @pl.when(extra)
    def _():
        pltpu.sync_copy(ii_hbm.at[pl.ds(base + nmain, CE)],
                        ii.at[pl.ds(nmain, CE)])
        pltpu.sync_copy(ij_hbm.at[pl.ds(base + nmain, CE)],
                        ij.at[pl.ds(nmain, CE)])

    def body(k, _):
        sl = pl.ds(k * L, L)
        a = ii[sl]
        b = ij[sl]
        dx = plsc.load_gather(rx, [a]) - plsc.load_gather(rx, [b])
        dy = plsc.load_gather(ry, [a]) - plsc.load_gather(ry, [b])
        dz = plsc.load_gather(rz, [a]) - plsc.load_gather(rz, [b])
        d2[sl] = dx * dx + dy * dy + dz * dz
        return 0

    nedge = nmain + jnp.where(extra, CE, 0)
    lax.fori_loop(0, nedge // L, body, 0)
    pltpu.sync_copy(d2.at[pl.ds(0, nmain)], d2_hbm.at[pl.ds(base, nmain)])

    @pl.when(extra)
    def _():
        pltpu.sync_copy(d2.at[pl.ds(nmain, CE)],
                        d2_hbm.at[pl.ds(base + nmain, CE)])


def _make_d2():
    mesh = plsc.VectorSubcoreMesh(core_axis_name="c", subcore_axis_name="s",
                                  num_cores=NC, num_subcores=NS)
    return pl.kernel(
        _d2_body,
        out_type=jax.ShapeDtypeStruct((E,), jnp.float32),
        mesh=mesh,
        scratch_types=[
            pltpu.VMEM((N,), jnp.float32),
            pltpu.VMEM((N,), jnp.float32),
            pltpu.VMEM((N,), jnp.float32),
            pltpu.VMEM((EPW_MAX,), jnp.int32),
            pltpu.VMEM((EPW_MAX,), jnp.int32),
            pltpu.VMEM((EPW_MAX,), jnp.float32),
        ],
        compiler_params=pltpu.CompilerParams(needs_layout_passes=False),
    )


def _seg_body(g3, hj, ii_hbm, ij_hbm, macc_hbm,
              idxi0, idxi1, idxj0, idxj1, hjg0, hjg1, gb0, gb1, xj0, xj1,
              acc, gsem0, gsem1, csem0, csem1, ssem0, ssem1,
              isem0, isem1, ksem0, ksem1):
    cid = lax.axis_index("c")
    sid = lax.axis_index("s")
    wid = sid * NC + cid
    idxis, idxjs = (idxi0, idxi1), (idxj0, idxj1)
    hjgs, gbs, xjs = (hjg0, hjg1), (gb0, gb1), (xj0, xj1)
    gsems, csems, ssems = (gsem0, gsem1), (csem0, csem1), (ssem0, ssem1)
    isems, ksems = (isem0, isem1), (ksem0, ksem1)

    base = (wid * PQ + jnp.minimum(wid, PR)) * 2
    npairs = PQ + jnp.where(wid < PR, 1, 0)

    def zrow(r, _):
        for c in range(FDIM // L):
            xj0[r, pl.ds(c * L, L)] = jnp.zeros((L,), jnp.float32)
        return 0

    lax.fori_loop(0, CE2, zrow, 0)
    r0 = sid * NPS
    for j0 in range(0, NPS - CE2 + 1, CE2):
        pltpu.sync_copy(xj0, acc.at[pl.ds(r0 + j0, CE2)])
    rem = NPS % CE2
    if rem:
        pltpu.sync_copy(xj0.at[pl.ds(0, rem)],
                        acc.at[pl.ds(r0 + NPS - rem, rem)])
    plsc.subcore_barrier()

    for b in range(2):
        pltpu.sync_copy(ij_hbm.at[pl.ds((base + b) * CE2, CE2)], idxjs[b])
        pltpu.async_copy(hj.at[idxjs[b]], hjgs[b], gsems[b])
        pltpu.async_copy(g3.at[base + b], gbs[b], csems[b])
        pltpu.async_copy(ii_hbm.at[pl.ds((base + b) * CE2, CE2)],
                         idxis[b], ksems[b])

    def pair(m, _):
        for b in range(2):
            k = base + 2 * m + b
            pltpu.make_async_copy(hj.at[idxjs[b]], hjgs[b], gsems[b]).wait()

            @pl.when(m + 1 < npairs)
            def _():
                pltpu.async_copy(ij_hbm.at[pl.ds((k + 2) * CE2, CE2)],
                                 idxjs[b], isems[b])

            pltpu.make_async_copy(g3.at[k], gbs[b], csems[b]).wait()

            @pl.when(m > 0)
            def _():
                pltpu.make_async_copy(xjs[b], acc.at[idxis[b]],
                                      ssems[b]).wait()
                pltpu.async_copy(ii_hbm.at[pl.ds(k * CE2, CE2)],
                                 idxis[b], ksems[b])

            def mrow(rr, _):
                for c in range(FDIM // L):
                    sl = pl.ds(c * L, L)
                    xjs[b][rr, sl] = hjgs[b][rr, sl] * gbs[b][rr, sl]
                return 0

            lax.fori_loop(0, CE2, mrow, 0)
            pltpu.make_async_copy(ii_hbm.at[pl.ds(k * CE2, CE2)],
                                  idxis[b], ksems[b]).wait()
            pltpu.async_copy(xjs[b], acc.at[idxis[b]], ssems[b], add=True)

            @pl.when(m + 1 < npairs)
            def _():
                pltpu.make_async_copy(ij_hbm.at[pl.ds((k + 2) * CE2, CE2)],
                                      idxjs[b], isems[b]).wait()
                pltpu.async_copy(hj.at[idxjs[b]], hjgs[b], gsems[b])
                pltpu.async_copy(g3.at[k + 2], gbs[b], csems[b])
        return 0

    lax.fori_loop(0, npairs, pair, 0)
    for b in range(2):
        pltpu.make_async_copy(xjs[b], acc.at[idxis[b]], ssems[b]).wait()
    plsc.subcore_barrier()
    pltpu.sync_copy(acc.at[pl.ds(r0, NPS)],
                    macc_hbm.at[cid, pl.ds(r0, NPS)])


def _make_seg():
    mesh = plsc.VectorSubcoreMesh(core_axis_name="c", subcore_axis_name="s",
                                  num_cores=NC, num_subcores=NS)
    return pl.kernel(
        _seg_body,
        out_type=jax.ShapeDtypeStruct((NC, NP, FDIM), jnp.float32),
        mesh=mesh,
        scratch_types=[
            pltpu.VMEM((CE2,), jnp.int32),
            pltpu.VMEM((CE2,), jnp.int32),
            pltpu.VMEM((CE2,), jnp.int32),
            pltpu.VMEM((CE2,), jnp.int32),
            pltpu.VMEM((CE2, FDIM), jnp.float32),
            pltpu.VMEM((CE2, FDIM), jnp.float32),
            pltpu.VMEM((CE2, FDIM), jnp.float32),
            pltpu.VMEM((CE2, FDIM), jnp.float32),
            pltpu.VMEM((CE2, FDIM), jnp.float32),
            pltpu.VMEM((CE2, FDIM), jnp.float32),
            pltpu.VMEM_SHARED((NP, FDIM), jnp.float32),
        ] + [pltpu.SemaphoreType.DMA] * 10,
        compiler_params=pltpu.CompilerParams(needs_layout_passes=False,
                                             use_tc_tiling_on_sc=False),
    )


def _g_body(d2_ref, cen_ref, wid_ref, k2f_ref, g0_ref, g1_ref, g2_ref,
            dij_ref):
    d2 = d2_ref[...]
    dij = jnp.sqrt(jnp.maximum(d2, 0.0))
    dij_ref[...] = dij
    xr = dij * (1.0 / SR_CUT)
    xr2 = xr * xr
    xr3 = xr2 * xr
    cut = 1.0 + ((15.0 - 6.0 * xr) * xr - 10.0) * xr3
    cut = jnp.where(dij < SR_CUT, cut, 0.0)
    mu = _softplus(cen_ref[...])
    beta = _softplus(wid_ref[...])
    diff = jnp.exp(-dij) - mu
    rbf = cut * jnp.exp(-beta * diff * diff)
    for b, ref in enumerate((g0_ref, g1_ref, g2_ref)):
        ref[...] = jnp.dot(rbf, k2f_ref[b],
                           preferred_element_type=jnp.float32)


def _run_g(d2c, centers_r, widths_r, k2fP):
    return pl.pallas_call(
        _g_body,
        grid=(E // GE,),
        in_specs=[
            pl.BlockSpec((GE, 1), lambda i: (i, 0)),
            pl.BlockSpec((1, K), lambda i: (0, 0)),
            pl.BlockSpec((1, K), lambda i: (0, 0)),
            pl.BlockSpec((NB, K, FDIM), lambda i: (0, 0, 0)),
        ],
        out_specs=[pl.BlockSpec((GE, FDIM), lambda i: (i, 0))] * 3 +
        [pl.BlockSpec((GE, 1), lambda i: (i, 0))],
        out_shape=[jax.ShapeDtypeStruct((E, FDIM), jnp.float32)] * 3 +
        [jax.ShapeDtypeStruct((E, 1), jnp.float32)],
    )(d2c, centers_r, widths_r, k2fP)


def _c0_body(z_ref, emb_ref, wi_ref, bi_ref, wj_ref, bj_ref,
             x_ref, xi_ref, hj_ref):
    zb = z_ref[...]
    iota = lax.broadcasted_iota(jnp.int32, (BN, FDIM), 1)
    oh = (iota == zb).astype(jnp.float32)
    x = jnp.dot(oh, emb_ref[...], preferred_element_type=jnp.float32)
    x_ref[...] = x
    xa = _ssp(x)
    xi_ref[...] = jnp.dot(xa, wi_ref[...],
                          preferred_element_type=jnp.float32) + bi_ref[...]
    hj_ref[...] = jnp.dot(xa, wj_ref[...],
                          preferred_element_type=jnp.float32) + bj_ref[...]


def _run_c0(z2, emb_pad, wi, bi, wj, bj):
    full = lambda shape: pl.BlockSpec(shape, lambda i: (0,) * len(shape))
    return pl.pallas_call(
        _c0_body,
        grid=(N // BN,),
        in_specs=[
            pl.BlockSpec((BN, 1), lambda i: (i, 0)),
            full((FDIM, FDIM)),
            full((FDIM, FDIM)),
            full((1, FDIM)),
            full((FDIM, FDIM)),
            full((1, FDIM)),
        ],
        out_specs=[pl.BlockSpec((BN, FDIM), lambda i: (i, 0))] * 3,
        out_shape=[jax.ShapeDtypeStruct((N, FDIM), jnp.float32)] * 3,
    )(z2, emb_pad, wi, bi, wj, bj)


def _node_body(has_next, xi_ref, macc_ref, x_ref,
               riW1_ref, rib1_ref, riW2_ref, rib2_ref,
               projW_ref, projb_ref, u_ref,
               raW1_ref, rab1_ref, raW2_ref, rab2_ref,
               roW1_ref, rob1_ref, roW2_ref, rob2_ref,
               outW_ref, outb_ref, wiN_ref, biN_ref, wjN_ref, bjN_ref,
               *out_refs):
    dot = functools.partial(jnp.dot, preferred_element_type=jnp.float32)
    m = xi_ref[...] + macc_ref[0] + macc_ref[1]
    for r in range(NRI):
        y = _ssp(dot(_ssp(m), riW1_ref[r]) + rib1_ref[r])
        m = m + dot(y, riW2_ref[r]) + rib2_ref[r]
    m = _ssp(m)
    x = u_ref[...] * x_ref[...] + dot(m, projW_ref[...]) + projb_ref[...]
    for r in range(NRA):
        y = _ssp(dot(_ssp(x), raW1_ref[r]) + rab1_ref[r])
        x = x + dot(y, raW2_ref[r]) + rab2_ref[r]
    o = x
    for r in range(NRO):
        y = _ssp(dot(_ssp(o), roW1_ref[r]) + rob1_ref[r])
        o = o + dot(y, roW2_ref[r]) + rob2_ref[r]
    out_refs[0][...] = dot(_ssp(o), outW_ref[...]) + outb_ref[...]
    if has_next:
        out_refs[1][...] = x
        xa = _ssp(x)
        out_refs[2][...] = dot(xa, wiN_ref[...]) + biN_ref[...]
        out_refs[3][...] = dot(xa, wjN_ref[...]) + bjN_ref[...]


def _run_node(has_next, xi, macc, x, wts):
    full = lambda shape: pl.BlockSpec(shape, lambda i: (0,) * len(shape))
    wspecs = [
        full((NRI, FDIM, FDIM)), full((NRI, 1, FDIM)),
        full((NRI, FDIM, FDIM)), full((NRI, 1, FDIM)),
        full((FDIM, FDIM)), full((1, FDIM)), full((1, FDIM)),
        full((NRA, FDIM, FDIM)), full((NRA, 1, FDIM)),
        full((NRA, FDIM, FDIM)), full((NRA, 1, FDIM)),
        full((NRO, FDIM, FDIM)), full((NRO, 1, FDIM)),
        full((NRO, FDIM, FDIM)), full((NRO, 1, FDIM)),
        full((FDIM, 2)), full((1, 2)),
        full((FDIM, FDIM)), full((1, FDIM)),
        full((FDIM, FDIM)), full((1, FDIM)),
    ]
    out_specs = [pl.BlockSpec((BN, 2), lambda i: (i, 0))]
    out_shape = [jax.ShapeDtypeStruct((N, 2), jnp.float32)]
    if has_next:
        out_specs += [pl.BlockSpec((BN, FDIM), lambda i: (i, 0))] * 3
        out_shape += [jax.ShapeDtypeStruct((N, FDIM), jnp.float32)] * 3
    return pl.pallas_call(
        functools.partial(_node_body, has_next),
        grid=(N // BN,),
        in_specs=[
            pl.BlockSpec((BN, FDIM), lambda i: (i, 0)),
            pl.BlockSpec((NC, BN, FDIM), lambda i: (0, i, 0)),
            pl.BlockSpec((BN, FDIM), lambda i: (i, 0)),
        ] + wspecs,
        out_specs=out_specs,
        out_shape=out_shape,
    )(xi, macc, x, *wts)


def _fin_body(z_ref, o0_ref, o1_ref, o2_ref,
              esc_ref, esh_ref, qsc_ref, qsh_ref,
              ea_ref, qa_ref, nh_ref):
    i = pl.program_id(0)
    zb = z_ref[...]
    iota = lax.broadcasted_iota(jnp.int32, (BN, FDIM), 1)
    oh = (iota == zb).astype(jnp.float32)
    dot = functools.partial(jnp.dot, preferred_element_type=jnp.float32)
    o0 = o0_ref[...]
    o1 = o1_ref[...]
    o2 = o2_ref[...]
    s = o0 + o1 + o2
    ea_ref[...] = dot(oh, esc_ref[...]) * s[:, 0:1] + dot(oh, esh_ref[...])
    qa_ref[...] = dot(oh, qsc_ref[...]) * s[:, 1:2] + dot(oh, qsh_ref[...])
    p0 = o0 * o0
    p1 = o1 * o1
    p2 = o2 * o2
    part = jnp.sum(p1 / (p1 + p0 + 1e-07)) + jnp.sum(p2 / (p2 + p1 + 1e-07))

    @pl.when(i == 0)
    def _():
        nh_ref[...] = jnp.zeros((1, 1), jnp.float32)

    nh_ref[...] += part * (1.0 / (2.0 * N))


def _run_fin(z2, o0, o1, o2, esc, esh, qsc, qsh):
    full = lambda shape: pl.BlockSpec(shape, lambda i: (0,) * len(shape))
    return pl.pallas_call(
        _fin_body,
        grid=(N // BN,),
        in_specs=[
            pl.BlockSpec((BN, 1), lambda i: (i, 0)),
            pl.BlockSpec((BN, 2), lambda i: (i, 0)),
            pl.BlockSpec((BN, 2), lambda i: (i, 0)),
            pl.BlockSpec((BN, 2), lambda i: (i, 0)),
            full((FDIM, 1)), full((FDIM, 1)), full((FDIM, 1)), full((FDIM, 1)),
        ],
        out_specs=[
            pl.BlockSpec((BN, 1), lambda i: (i, 0)),
            pl.BlockSpec((BN, 1), lambda i: (i, 0)),
            pl.BlockSpec((1, 1), lambda i: (0, 0)),
        ],
        out_shape=[
            jax.ShapeDtypeStruct((N, 1), jnp.float32),
            jax.ShapeDtypeStruct((N, 1), jnp.float32),
            jax.ShapeDtypeStruct((1, 1), jnp.float32),
        ],
    )(z2, o0, o1, o2, esc, esh, qsc, qsh)


def kernel(Z, R, idx_i, idx_j, params):
    p = params
    idx_i = idx_i.astype(jnp.int32)
    idx_j = idx_j.astype(jnp.int32)
    z2 = Z.astype(jnp.int32).reshape(N, 1)
    rx, ry, rz = R[:, 0], R[:, 1], R[:, 2]

    d2 = _make_d2()(rx, ry, rz, idx_i, idx_j)
    d2c = d2.reshape(E, 1)

    centers_r = p['centers'].reshape(1, K)
    widths_r = p['widths'].reshape(1, K)
    emb_pad = jnp.zeros((FDIM, FDIM), jnp.float32).at[:95].set(p['emb'])
    pad1 = lambda t: jnp.zeros((FDIM, 1), jnp.float32).at[:95, 0].set(t)

    seg = _make_seg()

    x, xi, hj = _run_c0(
        z2, emb_pad, p['Wi'][0], p['bi'][0].reshape(1, FDIM),
        p['Wj'][0], p['bj'][0].reshape(1, FDIM))

    gs0, gs1, gs2, dij_c = _run_g(d2c, centers_r, widths_r, p['k2f'])
    gs = (gs0, gs1, gs2)

    outs = []
    for b in range(NB):
        g3 = gs[b].reshape(NCH2, CE2, FDIM)
        macc = seg(g3, hj, idx_i, idx_j)
        has_next = b < NB - 1
        nb = b + 1 if has_next else 0
        wts = [
            p['riW1'][b], p['rib1'][b].reshape(NRI, 1, FDIM),
            p['riW2'][b], p['rib2'][b].reshape(NRI, 1, FDIM),
            p['projW'][b], p['projb'][b].reshape(1, FDIM),
            p['u'][b].reshape(1, FDIM),
            p['raW1'][b], p['rab1'][b].reshape(NRA, 1, FDIM),
            p['raW2'][b], p['rab2'][b].reshape(NRA, 1, FDIM),
            p['roW1'][b], p['rob1'][b].reshape(NRO, 1, FDIM),
            p['roW2'][b], p['rob2'][b].reshape(NRO, 1, FDIM),
            p['outW'][b], p['outb'][b].reshape(1, 2),
            p['Wi'][nb], p['bi'][nb].reshape(1, FDIM),
            p['Wj'][nb], p['bj'][nb].reshape(1, FDIM),
        ]
        res = _run_node(has_next, xi, macc, x, wts)
        if has_next:
            out_b, x, xi, hj = res
        else:
            (out_b,) = res
        outs.append(out_b)

    ea, qa, nh = _run_fin(z2, outs[0], outs[1], outs[2],
                          pad1(p['Escale']), pad1(p['Eshift']),
                          pad1(p['Qscale']), pad1(p['Qshift']))
    return (ea.reshape(N), qa.reshape(N), dij_c.reshape(E), nh.reshape(()))
